# Initial kernel scaffold; baseline (speedup 1.0000x reference)
import functools

import jax
import jax.numpy as jnp
import numpy as np
from jax.experimental import pallas as pl
from jax.experimental.pallas import tpu as pltpu

_K = 4
_BR = 256
_BC = 256
_BIG = jnp.float32(3.0e38)


def _knn_tile_kernel(collo_ref, ntiles_ref, hb_ref, sqr_ref, sqc_ref,
                     batr_ref, batc_ref, out_ref):
    pid = pl.program_id(0)
    r0 = pid * _BR
    col_lo = collo_ref[pid]
    nt = ntiles_ref[pid]
    A = hb_ref[pl.ds(r0, _BR), :]
    sqr = sqr_ref[pl.ds(r0, _BR), :]
    batr = batr_ref[pl.ds(r0, _BR), :]
    rows = r0 + jax.lax.broadcasted_iota(jnp.int32, (_BR, _BC), 0)

    def tile_body(t, carry):
        d0, d1, d2, d3, i0, i1, i2, i3 = carry
        c0 = col_lo + t * _BC
        B = hb_ref[pl.ds(c0, _BC), :]
        sqc = sqc_ref[:, pl.ds(c0, _BC)]
        batc = batc_ref[:, pl.ds(c0, _BC)]
        prod = jax.lax.dot_general(A, B, (((1,), (1,)), ((), ())),
                                   preferred_element_type=jnp.float32)
        d = sqr + sqc - 2.0 * prod
        cols = c0 + jax.lax.broadcasted_iota(jnp.int32, (_BR, _BC), 1)
        ok = (batr == batc) & (rows != cols)
        d = jnp.where(ok, d, _BIG)
        for _ in range(_K):
            m = jnp.min(d, axis=1, keepdims=True)
            marg = jnp.min(jnp.where(d == m, cols, jnp.int32(2**31 - 1)),
                           axis=1, keepdims=True)
            d = jnp.where(cols == marg, _BIG, d)
            b0 = m < d0
            b1 = m < d1
            b2 = m < d2
            b3 = m < d3
            d0n = jnp.where(b0, m, d0)
            i0n = jnp.where(b0, marg, i0)
            d1n = jnp.where(b0, d0, jnp.where(b1, m, d1))
            i1n = jnp.where(b0, i0, jnp.where(b1, marg, i1))
            d2n = jnp.where(b1, d1, jnp.where(b2, m, d2))
            i2n = jnp.where(b1, i1, jnp.where(b2, marg, i2))
            d3n = jnp.where(b2, d2, jnp.where(b3, m, d3))
            i3n = jnp.where(b2, i2, jnp.where(b3, marg, i3))
            d0, d1, d2, d3 = d0n, d1n, d2n, d3n
            i0, i1, i2, i3 = i0n, i1n, i2n, i3n
        return (d0, d1, d2, d3, i0, i1, i2, i3)

    big = jnp.full((_BR, 1), _BIG, jnp.float32)
    init = (big, big, big, big,
            jnp.zeros((_BR, 1), jnp.int32),
            jnp.full((_BR, 1), 1, jnp.int32),
            jnp.full((_BR, 1), 2, jnp.int32),
            jnp.full((_BR, 1), 3, jnp.int32))
    d0, d1, d2, d3, i0, i1, i2, i3 = jax.lax.fori_loop(0, nt, tile_body, init)
    out_ref[...] = jnp.concatenate([i0, i1, i2, i3], axis=1)


def _knn_idx(h, batch):
    n, f = h.shape
    nb = -(-n // _BR)
    n_pad = nb * _BR
    hb = jnp.zeros((n_pad, f), jnp.bfloat16).at[:n].set(h.astype(jnp.bfloat16))
    sq = jnp.sum(h * h, axis=1)
    sqr = jnp.zeros((n_pad, 1), jnp.float32).at[:n, 0].set(sq)
    sqc = sqr.reshape(1, n_pad)
    batr = jnp.full((n_pad, 1), -1, jnp.int32).at[:n, 0].set(batch)
    batc = jnp.full((1, n_pad), -2, jnp.int32).at[0, :n].set(batch)

    starts = jnp.arange(nb, dtype=jnp.int32) * _BR
    ends = jnp.minimum(starts + _BR - 1, n - 1)
    g_lo = batch[starts]
    g_hi = batch[ends]
    col_lo = jnp.searchsorted(batch, g_lo, side="left").astype(jnp.int32)
    col_hi = jnp.searchsorted(batch, g_hi, side="right").astype(jnp.int32)
    col_lo = (col_lo // _BC) * _BC
    ntiles = -((col_lo - col_hi) // _BC)

    grid_spec = pltpu.PrefetchScalarGridSpec(
        num_scalar_prefetch=2,
        grid=(nb,),
        in_specs=[
            pl.BlockSpec((n_pad, f), lambda i, *_: (0, 0)),
            pl.BlockSpec((n_pad, 1), lambda i, *_: (0, 0)),
            pl.BlockSpec((1, n_pad), lambda i, *_: (0, 0)),
            pl.BlockSpec((n_pad, 1), lambda i, *_: (0, 0)),
            pl.BlockSpec((1, n_pad), lambda i, *_: (0, 0)),
        ],
        out_specs=pl.BlockSpec((_BR, _K), lambda i, *_: (i, 0)),
    )
    idx = pl.pallas_call(
        _knn_tile_kernel,
        grid_spec=grid_spec,
        out_shape=jax.ShapeDtypeStruct((n_pad, _K), jnp.int32),
    )(col_lo, ntiles, hb, sqr, sqc, batr, batc)
    return idx[:n]


def _edge_conv(x, src, dst, Ws, bs):
    x_i = x[dst]
    x_j = x[src]
    m = jnp.concatenate([x_i, x_j - x_i], axis=1)
    for W, b in zip(Ws, bs):
        m = jax.nn.relu(m @ W + b)
    return jnp.zeros((x.shape[0], m.shape[1]), dtype=m.dtype).at[dst].add(m)


def kernel(x, edge_index, batch,
           W_c0_0, b_c0_0, W_c0_1, b_c0_1,
           W_c1_0, b_c1_0, W_c1_1, b_c1_1,
           W_c2_0, b_c2_0, W_c2_1, b_c2_1,
           W_c3_0, b_c3_0, W_c3_1, b_c3_1,
           W_p0, b_p0, W_p1, b_p1,
           W_h, b_h):
    n = x.shape[0]
    conv_w = [(W_c0_0, b_c0_0, W_c0_1, b_c0_1),
              (W_c1_0, b_c1_0, W_c1_1, b_c1_1),
              (W_c2_0, b_c2_0, W_c2_1, b_c2_1),
              (W_c3_0, b_c3_0, W_c3_1, b_c3_1)]
    skips = [x]
    h = x
    src = edge_index[0]
    dst = edge_index[1]
    for li, (W0, b0, W1, b1) in enumerate(conv_w):
        h = _edge_conv(h, src, dst, [W0, W1], [b0, b1])
        if li < 3:
            idx = _knn_idx(h, batch)
            src = idx.reshape(-1)
            dst = jnp.repeat(jnp.arange(n, dtype=jnp.int32), _K)
        skips.append(h)
    z = jnp.concatenate(skips, axis=1)
    z = jax.nn.relu(z @ W_p0 + b_p0)
    z = jax.nn.relu(z @ W_p1 + b_p1)
    z = z @ W_h + b_h
    return z


# speedup vs baseline: 3.4522x; 3.4522x over previous
import functools

import jax
import jax.numpy as jnp
import numpy as np
from jax.experimental import pallas as pl
from jax.experimental.pallas import tpu as pltpu

_K = 4
_BR = 256
_BC = 256
_BIG = 3.0e38


def _knn_tile_kernel(collo_ref, ntiles_ref, hb_ref, sqr_ref, sqc_ref,
                     batr_ref, batc_ref, out_ref):
    pid = pl.program_id(0)
    r0 = pl.multiple_of(pid * _BR, _BR)
    col_lo = pl.multiple_of(collo_ref[pid], _BC)
    nt = ntiles_ref[pid]
    A = hb_ref[pl.ds(r0, _BR), :]
    sqr = sqr_ref[pl.ds(r0, _BR), :]
    batr = batr_ref[pl.ds(r0, _BR), :]
    rows = r0 + jax.lax.broadcasted_iota(jnp.int32, (_BR, _BC), 0)

    def tile_body(t, carry):
        d0, d1, d2, d3, i0, i1, i2, i3 = carry
        c0 = pl.multiple_of(col_lo + t * _BC, _BC)
        B = hb_ref[pl.ds(c0, _BC), :]
        sqc = sqc_ref[:, pl.ds(c0, _BC)]
        batc = batc_ref[:, pl.ds(c0, _BC)]
        prod = jax.lax.dot_general(A, B, (((1,), (1,)), ((), ())),
                                   preferred_element_type=jnp.float32)
        d = sqr + sqc - 2.0 * prod
        cols = c0 + jax.lax.broadcasted_iota(jnp.int32, (_BR, _BC), 1)
        ok = (batr == batc) & (rows != cols)
        d = jnp.where(ok, d, _BIG)
        for _ in range(_K):
            m = jnp.min(d, axis=1, keepdims=True)
            marg = jnp.min(jnp.where(d == m, cols, jnp.int32(2**31 - 1)),
                           axis=1, keepdims=True)
            d = jnp.where(cols == marg, _BIG, d)
            b0 = m < d0
            b1 = m < d1
            b2 = m < d2
            b3 = m < d3
            d0n = jnp.where(b0, m, d0)
            i0n = jnp.where(b0, marg, i0)
            d1n = jnp.where(b0, d0, jnp.where(b1, m, d1))
            i1n = jnp.where(b0, i0, jnp.where(b1, marg, i1))
            d2n = jnp.where(b1, d1, jnp.where(b2, m, d2))
            i2n = jnp.where(b1, i1, jnp.where(b2, marg, i2))
            d3n = jnp.where(b2, d2, jnp.where(b3, m, d3))
            i3n = jnp.where(b2, i2, jnp.where(b3, marg, i3))
            d0, d1, d2, d3 = d0n, d1n, d2n, d3n
            i0, i1, i2, i3 = i0n, i1n, i2n, i3n
        return (d0, d1, d2, d3, i0, i1, i2, i3)

    big = jnp.full((_BR, 1), _BIG, jnp.float32)
    init = (big, big, big, big,
            jnp.zeros((_BR, 1), jnp.int32),
            jnp.full((_BR, 1), 1, jnp.int32),
            jnp.full((_BR, 1), 2, jnp.int32),
            jnp.full((_BR, 1), 3, jnp.int32))
    d0, d1, d2, d3, i0, i1, i2, i3 = jax.lax.fori_loop(0, nt, tile_body, init)
    out_ref[...] = jnp.concatenate([i0, i1, i2, i3], axis=1)


def _knn_idx(h, batch):
    n, f = h.shape
    nb = -(-n // _BR)
    n_pad = nb * _BR
    hb = jnp.zeros((n_pad, f), jnp.bfloat16).at[:n].set(h.astype(jnp.bfloat16))
    sq = jnp.sum(h * h, axis=1)
    sqr = jnp.zeros((n_pad, 1), jnp.float32).at[:n, 0].set(sq)
    sqc = sqr.reshape(1, n_pad)
    batr = jnp.full((n_pad, 1), -1, jnp.int32).at[:n, 0].set(batch)
    batc = jnp.full((1, n_pad), -2, jnp.int32).at[0, :n].set(batch)

    starts = jnp.arange(nb, dtype=jnp.int32) * _BR
    ends = jnp.minimum(starts + _BR - 1, n - 1)
    g_lo = batch[starts]
    g_hi = batch[ends]
    col_lo = jnp.searchsorted(batch, g_lo, side="left").astype(jnp.int32)
    col_hi = jnp.searchsorted(batch, g_hi, side="right").astype(jnp.int32)
    col_lo = (col_lo // _BC) * _BC
    ntiles = -((col_lo - col_hi) // _BC)

    grid_spec = pltpu.PrefetchScalarGridSpec(
        num_scalar_prefetch=2,
        grid=(nb,),
        in_specs=[
            pl.BlockSpec((n_pad, f), lambda i, *_: (0, 0)),
            pl.BlockSpec((n_pad, 1), lambda i, *_: (0, 0)),
            pl.BlockSpec((1, n_pad), lambda i, *_: (0, 0)),
            pl.BlockSpec((n_pad, 1), lambda i, *_: (0, 0)),
            pl.BlockSpec((1, n_pad), lambda i, *_: (0, 0)),
        ],
        out_specs=pl.BlockSpec((_BR, _K), lambda i, *_: (i, 0)),
    )
    idx = pl.pallas_call(
        _knn_tile_kernel,
        grid_spec=grid_spec,
        out_shape=jax.ShapeDtypeStruct((n_pad, _K), jnp.int32),
    )(col_lo, ntiles, hb, sqr, sqc, batr, batc)
    return idx[:n]


def _conv_knn_kernel(m4_ref, w1_ref, b1_ref, w2_ref, b2_ref, out_ref):
    acc = jnp.zeros(out_ref.shape, jnp.float32)
    for k in range(_K):
        mk = m4_ref[k].astype(jnp.bfloat16)
        h1 = jax.lax.dot_general(mk, w1_ref[...], (((1,), (0,)), ((), ())),
                                 preferred_element_type=jnp.float32)
        h1 = jnp.maximum(h1 + b1_ref[...], 0.0)
        h2 = jax.lax.dot_general(h1.astype(jnp.bfloat16), w2_ref[...],
                                 (((1,), (0,)), ((), ())),
                                 preferred_element_type=jnp.float32)
        acc = acc + jnp.maximum(h2 + b2_ref[...], 0.0)
    out_ref[...] = acc


def _conv_knn(m4, W1, b1, W2, b2):
    _, n_pad, f2 = m4.shape
    dmid = W1.shape[1]
    dout = W2.shape[1]
    bn = 256
    nb = n_pad // bn
    return pl.pallas_call(
        _conv_knn_kernel,
        grid=(nb,),
        in_specs=[
            pl.BlockSpec((_K, bn, f2), lambda i: (0, i, 0)),
            pl.BlockSpec((f2, dmid), lambda i: (0, 0)),
            pl.BlockSpec((1, dmid), lambda i: (0, 0)),
            pl.BlockSpec((dmid, dout), lambda i: (0, 0)),
            pl.BlockSpec((1, dout), lambda i: (0, 0)),
        ],
        out_specs=pl.BlockSpec((bn, dout), lambda i: (i, 0)),
        out_shape=jax.ShapeDtypeStruct((n_pad, dout), jnp.float32),
    )(m4, W1.astype(jnp.bfloat16), b1.reshape(1, -1),
      W2.astype(jnp.bfloat16), b2.reshape(1, -1))


def _conv0_mlp_kernel(m_ref, w1_ref, b1_ref, w2_ref, b2_ref, out_ref):
    mk = m_ref[...].astype(jnp.bfloat16)
    h1 = jax.lax.dot_general(mk, w1_ref[...], (((1,), (0,)), ((), ())),
                             preferred_element_type=jnp.float32)
    h1 = jnp.maximum(h1 + b1_ref[...], 0.0)
    h2 = jax.lax.dot_general(h1.astype(jnp.bfloat16), w2_ref[...],
                             (((1,), (0,)), ((), ())),
                             preferred_element_type=jnp.float32)
    out_ref[...] = jnp.maximum(h2 + b2_ref[...], 0.0)


def _conv0_mlp(m, W1, b1, W2, b2):
    e, f2 = m.shape
    dmid = W1.shape[1]
    dout = W2.shape[1]
    be = 512
    ne = e // be
    return pl.pallas_call(
        _conv0_mlp_kernel,
        grid=(ne,),
        in_specs=[
            pl.BlockSpec((be, f2), lambda i: (i, 0)),
            pl.BlockSpec((f2, dmid), lambda i: (0, 0)),
            pl.BlockSpec((1, dmid), lambda i: (0, 0)),
            pl.BlockSpec((dmid, dout), lambda i: (0, 0)),
            pl.BlockSpec((1, dout), lambda i: (0, 0)),
        ],
        out_specs=pl.BlockSpec((be, dout), lambda i: (i, 0)),
        out_shape=jax.ShapeDtypeStruct((e, dout), jnp.float32),
    )(m, W1.astype(jnp.bfloat16), b1.reshape(1, -1),
      W2.astype(jnp.bfloat16), b2.reshape(1, -1))


def _post_mlp_kernel(x_ref, h1_ref, h2_ref, h3_ref, h4_ref,
                     wp0a_ref, wp0b_ref, wp0c_ref, wp0d_ref, wp0e_ref,
                     bp0_ref, wp1_ref, bp1_ref, wh_ref, bh_ref, out_ref):
    def bdot(a, w):
        return jax.lax.dot_general(a.astype(jnp.bfloat16), w,
                                   (((1,), (0,)), ((), ())),
                                   preferred_element_type=jnp.float32)
    z = (bdot(x_ref[...], wp0a_ref[...]) + bdot(h1_ref[...], wp0b_ref[...])
         + bdot(h2_ref[...], wp0c_ref[...]) + bdot(h3_ref[...], wp0d_ref[...])
         + bdot(h4_ref[...], wp0e_ref[...]))
    z = jnp.maximum(z + bp0_ref[...], 0.0)
    z = jnp.maximum(bdot(z, wp1_ref[...]) + bp1_ref[...], 0.0)
    out_ref[...] = bdot(z, wh_ref[...]) + bh_ref[...]


def _post_mlp(x, hs, W_p0, b_p0, W_p1, b_p1, W_h, b_h):
    n = x.shape[0]
    f = x.shape[1]
    bn = 512
    nb = -(-n // bn)
    n_pad = nb * bn
    fh = hs[0].shape[1]
    xp = jnp.zeros((n_pad, f), jnp.float32).at[:n].set(x)
    hps = [jnp.zeros((n_pad, fh), jnp.float32).at[:n].set(h) for h in hs]
    wb = W_p0.astype(jnp.bfloat16)
    w_splits = [wb[0:f]]
    off = f
    for _ in range(4):
        w_splits.append(wb[off:off + fh])
        off += fh
    dout = W_h.shape[1]
    out = pl.pallas_call(
        _post_mlp_kernel,
        grid=(nb,),
        in_specs=[
            pl.BlockSpec((bn, f), lambda i: (i, 0)),
            pl.BlockSpec((bn, fh), lambda i: (i, 0)),
            pl.BlockSpec((bn, fh), lambda i: (i, 0)),
            pl.BlockSpec((bn, fh), lambda i: (i, 0)),
            pl.BlockSpec((bn, fh), lambda i: (i, 0)),
            pl.BlockSpec(w_splits[0].shape, lambda i: (0, 0)),
            pl.BlockSpec(w_splits[1].shape, lambda i: (0, 0)),
            pl.BlockSpec(w_splits[2].shape, lambda i: (0, 0)),
            pl.BlockSpec(w_splits[3].shape, lambda i: (0, 0)),
            pl.BlockSpec(w_splits[4].shape, lambda i: (0, 0)),
            pl.BlockSpec((1, W_p0.shape[1]), lambda i: (0, 0)),
            pl.BlockSpec(W_p1.shape, lambda i: (0, 0)),
            pl.BlockSpec((1, W_p1.shape[1]), lambda i: (0, 0)),
            pl.BlockSpec(W_h.shape, lambda i: (0, 0)),
            pl.BlockSpec((1, dout), lambda i: (0, 0)),
        ],
        out_specs=pl.BlockSpec((bn, dout), lambda i: (i, 0)),
        out_shape=jax.ShapeDtypeStruct((n_pad, dout), jnp.float32),
    )(xp, *hps, *w_splits, b_p0.reshape(1, -1),
      W_p1.astype(jnp.bfloat16), b_p1.reshape(1, -1),
      W_h.astype(jnp.bfloat16), b_h.reshape(1, -1))
    return out[:n]


def kernel(x, edge_index, batch,
           W_c0_0, b_c0_0, W_c0_1, b_c0_1,
           W_c1_0, b_c1_0, W_c1_1, b_c1_1,
           W_c2_0, b_c2_0, W_c2_1, b_c2_1,
           W_c3_0, b_c3_0, W_c3_1, b_c3_1,
           W_p0, b_p0, W_p1, b_p1,
           W_h, b_h):
    n = x.shape[0]
    src = edge_index[0]
    dst = edge_index[1]
    x_i = x[dst]
    x_j = x[src]
    m0 = jnp.concatenate([x_i, x_j - x_i], axis=1)
    e2 = _conv0_mlp(m0, W_c0_0, b_c0_0, W_c0_1, b_c0_1)
    h = jnp.zeros((n, e2.shape[1]), jnp.float32).at[dst].add(e2)

    skips = [x, h]
    knn_w = [(W_c1_0, b_c1_0, W_c1_1, b_c1_1),
             (W_c2_0, b_c2_0, W_c2_1, b_c2_1),
             (W_c3_0, b_c3_0, W_c3_1, b_c3_1)]
    bn = 256
    n_pad = (-(-n // bn)) * bn
    for W0, b0, W1, b1 in knn_w:
        idx = _knn_idx(h, batch)
        fh = h.shape[1]
        m4 = jnp.zeros((_K, n_pad, 2 * fh), jnp.float32)
        for k in range(_K):
            hj = h[idx[:, k]]
            m4 = m4.at[k, :n, :fh].set(h)
            m4 = m4.at[k, :n, fh:].set(hj - h)
        h = _conv_knn(m4, W0, b0, W1, b1)[:n]
        skips.append(h)

    return _post_mlp(x, skips[1:], W_p0, b_p0, W_p1, b_p1, W_h, b_h)


# speedup vs baseline: 4.5432x; 1.3160x over previous
import functools

import jax
import jax.numpy as jnp
import numpy as np
from jax.experimental import pallas as pl
from jax.experimental.pallas import tpu as pltpu
from jax.experimental.pallas import tpu_sc as plsc

_K = 4
_BR = 256
_BC = 256
_BIG = 3.0e38


def _knn_tile_kernel(collo_ref, ntiles_ref, hb_ref, sqr_ref, sqc_ref,
                     batr_ref, batc_ref, out_ref):
    pid = pl.program_id(0)
    r0 = pl.multiple_of(pid * _BR, _BR)
    col_lo = pl.multiple_of(collo_ref[pid], _BC)
    nt = ntiles_ref[pid]
    A = hb_ref[pl.ds(r0, _BR), :]
    sqr = sqr_ref[pl.ds(r0, _BR), :]
    batr = batr_ref[pl.ds(r0, _BR), :]
    rows = r0 + jax.lax.broadcasted_iota(jnp.int32, (_BR, _BC), 0)

    def tile_body(t, carry):
        d0, d1, d2, d3, i0, i1, i2, i3 = carry
        c0 = pl.multiple_of(col_lo + t * _BC, _BC)
        B = hb_ref[pl.ds(c0, _BC), :]
        sqc = sqc_ref[:, pl.ds(c0, _BC)]
        batc = batc_ref[:, pl.ds(c0, _BC)]
        prod = jax.lax.dot_general(A, B, (((1,), (1,)), ((), ())),
                                   preferred_element_type=jnp.float32)
        d = sqr + sqc - 2.0 * prod
        cols = c0 + jax.lax.broadcasted_iota(jnp.int32, (_BR, _BC), 1)
        ok = (batr == batc) & (rows != cols)
        d = jnp.where(ok, d, _BIG)
        for _ in range(_K):
            m = jnp.min(d, axis=1, keepdims=True)
            marg = jnp.min(jnp.where(d == m, cols, jnp.int32(2**31 - 1)),
                           axis=1, keepdims=True)
            d = jnp.where(cols == marg, _BIG, d)
            b0 = m < d0
            b1 = m < d1
            b2 = m < d2
            b3 = m < d3
            d0n = jnp.where(b0, m, d0)
            i0n = jnp.where(b0, marg, i0)
            d1n = jnp.where(b0, d0, jnp.where(b1, m, d1))
            i1n = jnp.where(b0, i0, jnp.where(b1, marg, i1))
            d2n = jnp.where(b1, d1, jnp.where(b2, m, d2))
            i2n = jnp.where(b1, i1, jnp.where(b2, marg, i2))
            d3n = jnp.where(b2, d2, jnp.where(b3, m, d3))
            i3n = jnp.where(b2, i2, jnp.where(b3, marg, i3))
            d0, d1, d2, d3 = d0n, d1n, d2n, d3n
            i0, i1, i2, i3 = i0n, i1n, i2n, i3n
        return (d0, d1, d2, d3, i0, i1, i2, i3)

    big = jnp.full((_BR, 1), _BIG, jnp.float32)
    init = (big, big, big, big,
            jnp.zeros((_BR, 1), jnp.int32),
            jnp.full((_BR, 1), 1, jnp.int32),
            jnp.full((_BR, 1), 2, jnp.int32),
            jnp.full((_BR, 1), 3, jnp.int32))
    d0, d1, d2, d3, i0, i1, i2, i3 = jax.lax.fori_loop(0, nt, tile_body, init)
    out_ref[...] = jnp.concatenate([i0, i1, i2, i3], axis=1)


def _knn_idx(h, batch):
    n, f = h.shape
    nb = -(-n // _BR)
    n_pad = nb * _BR
    hb = jnp.zeros((n_pad, f), jnp.bfloat16).at[:n].set(h.astype(jnp.bfloat16))
    sq = jnp.sum(h * h, axis=1)
    sqr = jnp.zeros((n_pad, 1), jnp.float32).at[:n, 0].set(sq)
    sqc = sqr.reshape(1, n_pad)
    batr = jnp.full((n_pad, 1), -1, jnp.int32).at[:n, 0].set(batch)
    batc = jnp.full((1, n_pad), -2, jnp.int32).at[0, :n].set(batch)

    starts = jnp.arange(nb, dtype=jnp.int32) * _BR
    ends = jnp.minimum(starts + _BR - 1, n - 1)
    g_lo = batch[starts]
    g_hi = batch[ends]
    col_lo = jnp.searchsorted(batch, g_lo, side="left").astype(jnp.int32)
    col_hi = jnp.searchsorted(batch, g_hi, side="right").astype(jnp.int32)
    col_lo = (col_lo // _BC) * _BC
    ntiles = -((col_lo - col_hi) // _BC)

    grid_spec = pltpu.PrefetchScalarGridSpec(
        num_scalar_prefetch=2,
        grid=(nb,),
        in_specs=[
            pl.BlockSpec((n_pad, f), lambda i, *_: (0, 0)),
            pl.BlockSpec((n_pad, 1), lambda i, *_: (0, 0)),
            pl.BlockSpec((1, n_pad), lambda i, *_: (0, 0)),
            pl.BlockSpec((n_pad, 1), lambda i, *_: (0, 0)),
            pl.BlockSpec((1, n_pad), lambda i, *_: (0, 0)),
        ],
        out_specs=pl.BlockSpec((_BR, _K), lambda i, *_: (i, 0)),
    )
    idx = pl.pallas_call(
        _knn_tile_kernel,
        grid_spec=grid_spec,
        out_shape=jax.ShapeDtypeStruct((n_pad, _K), jnp.int32),
    )(col_lo, ntiles, hb, sqr, sqc, batr, batc)
    return idx[:n]


def _sc_scatter_kernel(e2_hbm, dst_hbm, zeros_hbm, out_hbm,
                       idx_v, idxt_v, slab_v, acc):
    c = jax.lax.axis_index("c")
    s = jax.lax.axis_index("s")
    e = e2_hbm.shape[0]
    n_pad = out_hbm.shape[0]
    rows_per_tile = n_pad // 16
    r0 = s * rows_per_tile
    per_tile = e // 16
    n_chunks = per_tile // 128
    tail = per_tile - n_chunks * 128
    e_base = s * per_tile
    for p in range(2):
        fo = (c * 2 + p) * 128
        pltpu.sync_copy(zeros_hbm.at[pl.ds(r0, rows_per_tile), :],
                        acc.at[pl.ds(r0, rows_per_tile), :])
        plsc.subcore_barrier()

        def chunk(j, carry):
            e0 = e_base + j * 128
            pltpu.sync_copy(dst_hbm.at[pl.ds(e0, 128)], idx_v)
            pltpu.sync_copy(e2_hbm.at[pl.ds(e0, 128), pl.ds(fo, 128)], slab_v)
            pltpu.sync_copy(slab_v, acc.at[idx_v], add=True)
            return carry

        jax.lax.fori_loop(0, n_chunks, chunk, 0)
        if tail:
            e0 = e_base + n_chunks * 128
            pltpu.sync_copy(dst_hbm.at[pl.ds(e0, tail)], idxt_v)
            pltpu.sync_copy(e2_hbm.at[pl.ds(e0, tail), pl.ds(fo, 128)],
                            slab_v.at[pl.ds(0, tail), :])
            pltpu.sync_copy(slab_v.at[pl.ds(0, tail), :], acc.at[idxt_v],
                            add=True)
        plsc.subcore_barrier()
        pltpu.sync_copy(acc.at[pl.ds(r0, rows_per_tile), :],
                        out_hbm.at[pl.ds(r0, rows_per_tile), pl.ds(fo, 128)])
        plsc.subcore_barrier()


def _sc_scatter_add(e2, dst, n):
    n_pad = 10240
    dout = e2.shape[1]
    mesh = plsc.VectorSubcoreMesh(core_axis_name="c", subcore_axis_name="s")
    tail = (e2.shape[0] // 16) % 128
    f = pl.kernel(
        _sc_scatter_kernel,
        out_type=jax.ShapeDtypeStruct((n_pad, dout), jnp.float32),
        mesh=mesh,
        scratch_types=[
            pltpu.VMEM((128,), jnp.int32),
            pltpu.VMEM((max(tail, 8),), jnp.int32),
            pltpu.VMEM((128, 128), jnp.float32),
            pltpu.VMEM_SHARED((n_pad, 128), jnp.float32),
        ],
    )
    out = f(e2, dst, jnp.zeros((n_pad, 128), jnp.float32))
    return out[:n]


def _conv_knn_kernel(m4_ref, w1_ref, b1_ref, w2_ref, b2_ref, out_ref):
    acc = jnp.zeros(out_ref.shape, jnp.float32)
    for k in range(_K):
        mk = m4_ref[k].astype(jnp.bfloat16)
        h1 = jax.lax.dot_general(mk, w1_ref[...], (((1,), (0,)), ((), ())),
                                 preferred_element_type=jnp.float32)
        h1 = jnp.maximum(h1 + b1_ref[...], 0.0)
        h2 = jax.lax.dot_general(h1.astype(jnp.bfloat16), w2_ref[...],
                                 (((1,), (0,)), ((), ())),
                                 preferred_element_type=jnp.float32)
        acc = acc + jnp.maximum(h2 + b2_ref[...], 0.0)
    out_ref[...] = acc


def _conv_knn(m4, W1, b1, W2, b2):
    _, n_pad, f2 = m4.shape
    dmid = W1.shape[1]
    dout = W2.shape[1]
    bn = 256
    nb = n_pad // bn
    return pl.pallas_call(
        _conv_knn_kernel,
        grid=(nb,),
        in_specs=[
            pl.BlockSpec((_K, bn, f2), lambda i: (0, i, 0)),
            pl.BlockSpec((f2, dmid), lambda i: (0, 0)),
            pl.BlockSpec((1, dmid), lambda i: (0, 0)),
            pl.BlockSpec((dmid, dout), lambda i: (0, 0)),
            pl.BlockSpec((1, dout), lambda i: (0, 0)),
        ],
        out_specs=pl.BlockSpec((bn, dout), lambda i: (i, 0)),
        out_shape=jax.ShapeDtypeStruct((n_pad, dout), jnp.float32),
    )(m4, W1.astype(jnp.bfloat16), b1.reshape(1, -1),
      W2.astype(jnp.bfloat16), b2.reshape(1, -1))


def _conv0_mlp_kernel(m_ref, w1_ref, b1_ref, w2_ref, b2_ref, out_ref):
    mk = m_ref[...].astype(jnp.bfloat16)
    h1 = jax.lax.dot_general(mk, w1_ref[...], (((1,), (0,)), ((), ())),
                             preferred_element_type=jnp.float32)
    h1 = jnp.maximum(h1 + b1_ref[...], 0.0)
    h2 = jax.lax.dot_general(h1.astype(jnp.bfloat16), w2_ref[...],
                             (((1,), (0,)), ((), ())),
                             preferred_element_type=jnp.float32)
    out_ref[...] = jnp.maximum(h2 + b2_ref[...], 0.0)


def _conv0_mlp(m, W1, b1, W2, b2):
    e, f2 = m.shape
    dmid = W1.shape[1]
    dout = W2.shape[1]
    be = 512
    ne = e // be
    return pl.pallas_call(
        _conv0_mlp_kernel,
        grid=(ne,),
        in_specs=[
            pl.BlockSpec((be, f2), lambda i: (i, 0)),
            pl.BlockSpec((f2, dmid), lambda i: (0, 0)),
            pl.BlockSpec((1, dmid), lambda i: (0, 0)),
            pl.BlockSpec((dmid, dout), lambda i: (0, 0)),
            pl.BlockSpec((1, dout), lambda i: (0, 0)),
        ],
        out_specs=pl.BlockSpec((be, dout), lambda i: (i, 0)),
        out_shape=jax.ShapeDtypeStruct((e, dout), jnp.float32),
    )(m, W1.astype(jnp.bfloat16), b1.reshape(1, -1),
      W2.astype(jnp.bfloat16), b2.reshape(1, -1))


def _post_mlp_kernel(x_ref, h1_ref, h2_ref, h3_ref, h4_ref,
                     wp0a_ref, wp0b_ref, wp0c_ref, wp0d_ref, wp0e_ref,
                     bp0_ref, wp1_ref, bp1_ref, wh_ref, bh_ref, out_ref):
    def bdot(a, w):
        return jax.lax.dot_general(a.astype(jnp.bfloat16), w,
                                   (((1,), (0,)), ((), ())),
                                   preferred_element_type=jnp.float32)
    z = (bdot(x_ref[...], wp0a_ref[...]) + bdot(h1_ref[...], wp0b_ref[...])
         + bdot(h2_ref[...], wp0c_ref[...]) + bdot(h3_ref[...], wp0d_ref[...])
         + bdot(h4_ref[...], wp0e_ref[...]))
    z = jnp.maximum(z + bp0_ref[...], 0.0)
    z = jnp.maximum(bdot(z, wp1_ref[...]) + bp1_ref[...], 0.0)
    out_ref[...] = bdot(z, wh_ref[...]) + bh_ref[...]


def _post_mlp(x, hs, W_p0, b_p0, W_p1, b_p1, W_h, b_h):
    n = x.shape[0]
    f = x.shape[1]
    bn = 512
    nb = -(-n // bn)
    n_pad = nb * bn
    fh = hs[0].shape[1]
    xp = jnp.zeros((n_pad, f), jnp.float32).at[:n].set(x)
    hps = [jnp.zeros((n_pad, fh), jnp.float32).at[:n].set(h) for h in hs]
    wb = W_p0.astype(jnp.bfloat16)
    w_splits = [wb[0:f]]
    off = f
    for _ in range(4):
        w_splits.append(wb[off:off + fh])
        off += fh
    dout = W_h.shape[1]
    out = pl.pallas_call(
        _post_mlp_kernel,
        grid=(nb,),
        in_specs=[
            pl.BlockSpec((bn, f), lambda i: (i, 0)),
            pl.BlockSpec((bn, fh), lambda i: (i, 0)),
            pl.BlockSpec((bn, fh), lambda i: (i, 0)),
            pl.BlockSpec((bn, fh), lambda i: (i, 0)),
            pl.BlockSpec((bn, fh), lambda i: (i, 0)),
            pl.BlockSpec(w_splits[0].shape, lambda i: (0, 0)),
            pl.BlockSpec(w_splits[1].shape, lambda i: (0, 0)),
            pl.BlockSpec(w_splits[2].shape, lambda i: (0, 0)),
            pl.BlockSpec(w_splits[3].shape, lambda i: (0, 0)),
            pl.BlockSpec(w_splits[4].shape, lambda i: (0, 0)),
            pl.BlockSpec((1, W_p0.shape[1]), lambda i: (0, 0)),
            pl.BlockSpec(W_p1.shape, lambda i: (0, 0)),
            pl.BlockSpec((1, W_p1.shape[1]), lambda i: (0, 0)),
            pl.BlockSpec(W_h.shape, lambda i: (0, 0)),
            pl.BlockSpec((1, dout), lambda i: (0, 0)),
        ],
        out_specs=pl.BlockSpec((bn, dout), lambda i: (i, 0)),
        out_shape=jax.ShapeDtypeStruct((n_pad, dout), jnp.float32),
    )(xp, *hps, *w_splits, b_p0.reshape(1, -1),
      W_p1.astype(jnp.bfloat16), b_p1.reshape(1, -1),
      W_h.astype(jnp.bfloat16), b_h.reshape(1, -1))
    return out[:n]


def kernel(x, edge_index, batch,
           W_c0_0, b_c0_0, W_c0_1, b_c0_1,
           W_c1_0, b_c1_0, W_c1_1, b_c1_1,
           W_c2_0, b_c2_0, W_c2_1, b_c2_1,
           W_c3_0, b_c3_0, W_c3_1, b_c3_1,
           W_p0, b_p0, W_p1, b_p1,
           W_h, b_h):
    n = x.shape[0]
    src = edge_index[0]
    dst = edge_index[1]
    x_i = x[dst]
    x_j = x[src]
    m0 = jnp.concatenate([x_i, x_j - x_i], axis=1)
    e2 = _conv0_mlp(m0, W_c0_0, b_c0_0, W_c0_1, b_c0_1)
    h = _sc_scatter_add(e2, dst, n)

    skips = [x, h]
    knn_w = [(W_c1_0, b_c1_0, W_c1_1, b_c1_1),
             (W_c2_0, b_c2_0, W_c2_1, b_c2_1),
             (W_c3_0, b_c3_0, W_c3_1, b_c3_1)]
    bn = 256
    n_pad = (-(-n // bn)) * bn
    for W0, b0, W1, b1 in knn_w:
        idx = _knn_idx(h, batch)
        fh = h.shape[1]
        m4 = jnp.zeros((_K, n_pad, 2 * fh), jnp.float32)
        for k in range(_K):
            hj = h[idx[:, k]]
            m4 = m4.at[k, :n, :fh].set(h)
            m4 = m4.at[k, :n, fh:].set(hj - h)
        h = _conv_knn(m4, W0, b0, W1, b1)[:n]
        skips.append(h)

    return _post_mlp(x, skips[1:], W_p0, b_p0, W_p1, b_p1, W_h, b_h)
